# baseline (speedup 1.0000x reference)
import functools

import jax
import jax.numpy as jnp
from jax import lax
from jax.experimental import pallas as pl
from jax.experimental.pallas import tpu as pltpu
from jax.experimental.pallas import tpu_sc as plsc

H = W = 512
N = 2 * H * W
NQ = 8 * 3 * H * W
NW = 32
QPW = NQ // NW
CHUNK = 4096
NCHUNK = QPW // CHUNK
NVREG = CHUNK // 16
NSUB = CHUNK // 128
VPI = 4
NITER = NVREG // VPI
PRE = N // 8


def _sc_body(q_hbm, bnd_hbm, val_hbm, pre_hbm, out_hbm,
             pre_v, qbuf, lbuf, midb, cmpb, cmp2b, obuf, sbnd, sem):
    cid = lax.axis_index("c")
    sid = lax.axis_index("s")
    wid = sid * 2 + cid

    pltpu.sync_copy(pre_hbm, pre_v)

    @pl.when(sid == 0)
    def _stage():
        pltpu.sync_copy(bnd_hbm, sbnd)

    plsc.subcore_barrier()

    def gather_round(pairs):
        def fire(k, _):
            off = pl.multiple_of(k * 128, 128)
            for table, idx, dst in pairs:
                pltpu.async_copy(table.at[idx.at[pl.ds(off, 128)]],
                                 dst.at[pl.ds(off, 128)], sem)
            return 0
        lax.fori_loop(0, NSUB, fire, 0)
        for _, _, dst in pairs:
            pltpu.make_async_copy(q_hbm.at[pl.ds(0, CHUNK)], dst, sem).wait()

    def pass_prefix(j, _):
        offs = [pl.multiple_of(j * (16 * VPI) + u * 16, 16) for u in range(VPI)]
        qs = [qbuf[pl.ds(o, 16)] for o in offs]
        ls = [jnp.zeros((16,), jnp.int32) for _ in range(VPI)]
        for d in range(16):
            for u in range(VPI):
                i = (ls[u] >> 3) + jnp.int32(1 << (15 - d))
                t = plsc.load_gather(pre_v, [i])
                c = qs[u] > t
                ls[u] = ls[u] + jnp.where(c, jnp.int32(1 << (18 - d)),
                                          jnp.int32(0))
        for u in range(VPI):
            lbuf[pl.ds(offs[u], 16)] = ls[u]
            midb[pl.ds(offs[u], 16)] = ls[u] + jnp.int32(4)
        return 0

    def make_mid_pass(step, nxt):
        def body(j, _):
            for u in range(VPI):
                off = pl.multiple_of(j * (16 * VPI) + u * 16, 16)
                q = qbuf[pl.ds(off, 16)]
                l = lbuf[pl.ds(off, 16)]
                t = cmpb[pl.ds(off, 16)]
                l = l + jnp.where(q > t, jnp.int32(step), jnp.int32(0))
                lbuf[pl.ds(off, 16)] = l
                midb[pl.ds(off, 16)] = l + jnp.int32(nxt)
            return 0
        return body

    def pass_final(j, _):
        for u in range(VPI):
            off = pl.multiple_of(j * (16 * VPI) + u * 16, 16)
            q = qbuf[pl.ds(off, 16)]
            l = lbuf[pl.ds(off, 16)]
            t1 = cmpb[pl.ds(off, 16)]
            t0 = cmp2b[pl.ds(off, 16)]
            c = q > t1
            l = l + c.astype(jnp.int32)
            tf = jnp.where(c, t1, t0)
            res = l + (q > tf).astype(jnp.int32)
            midb[pl.ds(off, 16)] = jnp.minimum(res, jnp.int32(N - 1))
        return 0

    def chunk_body(ch, _):
        base = pl.multiple_of(wid * QPW + ch * CHUNK, CHUNK)
        pltpu.sync_copy(q_hbm.at[pl.ds(base, CHUNK)], qbuf)
        lax.fori_loop(0, NITER, pass_prefix, 0)
        gather_round([(sbnd, midb, cmpb)])
        lax.fori_loop(0, NITER, make_mid_pass(4, 2), 0)
        gather_round([(sbnd, midb, cmpb)])
        lax.fori_loop(0, NITER, make_mid_pass(2, 1), 0)
        gather_round([(sbnd, midb, cmpb),
                      (sbnd, lbuf, cmp2b)])
        lax.fori_loop(0, NITER, pass_final, 0)
        gather_round([(val_hbm, midb, obuf)])
        pltpu.sync_copy(obuf, out_hbm.at[pl.ds(base, CHUNK)])
        return 0

    lax.fori_loop(0, NCHUNK, chunk_body, 0)


@jax.jit
def kernel(image, yx_res):
    b, c, h, w = yx_res.shape
    xs = (jnp.arange(w, dtype=jnp.float32) / (w - 1)) * 2.0 - 1.0
    ys = (jnp.arange(h, dtype=jnp.float32) / (h - 1)) * 2.0 - 1.0
    xm = jnp.broadcast_to(xs[None, :], (h, w))
    ym = jnp.broadcast_to(ys[:, None], (h, w))
    bnd = jnp.stack([xm + yx_res[0, 0], ym + yx_res[0, 1]], axis=-1).ravel()
    val = jnp.stack([xm + yx_res[1, 0], ym + yx_res[1, 1]], axis=-1).ravel()
    pre = bnd.reshape(PRE, 8)[:, 0]
    qflat = image.ravel()

    mesh = plsc.VectorSubcoreMesh(core_axis_name="c", subcore_axis_name="s")
    out = pl.kernel(
        _sc_body,
        out_type=jax.ShapeDtypeStruct((NQ,), jnp.float32),
        mesh=mesh,
        compiler_params=pltpu.CompilerParams(needs_layout_passes=False),
        scratch_types=[
            pltpu.VMEM((PRE,), jnp.float32),
            pltpu.VMEM((CHUNK,), jnp.float32),
            pltpu.VMEM((CHUNK,), jnp.int32),
            pltpu.VMEM((CHUNK,), jnp.int32),
            pltpu.VMEM((CHUNK,), jnp.float32),
            pltpu.VMEM((CHUNK,), jnp.float32),
            pltpu.VMEM((CHUNK,), jnp.float32),
            pltpu.VMEM_SHARED((N,), jnp.float32),
            pltpu.SemaphoreType.DMA,
        ],
    )(qflat, bnd, val, pre)
    return out.reshape(image.shape)


# speedup vs baseline: 1.0018x; 1.0018x over previous
import functools

import jax
import jax.numpy as jnp
from jax import lax
from jax.experimental import pallas as pl
from jax.experimental.pallas import tpu as pltpu
from jax.experimental.pallas import tpu_sc as plsc

H = W = 512
N = 2 * H * W
NQ = 8 * 3 * H * W
NW = 32
QPW = NQ // NW
CHUNK = 4096
NCHUNK = QPW // CHUNK
NVREG = CHUNK // 16
VPI = 4
NITER = NVREG // VPI
PRE = N // 8


def _sc_body(q_hbm, bnd_hbm, val_hbm, pre_hbm, out_hbm,
             pre_v, qbuf, lbuf, midb, cmpb, cmp2b, obuf, sbnd, sem):
    cid = lax.axis_index("c")
    sid = lax.axis_index("s")
    wid = sid * 2 + cid

    pltpu.sync_copy(pre_hbm, pre_v)

    @pl.when(sid == 0)
    def _stage():
        pltpu.sync_copy(bnd_hbm, sbnd)

    plsc.subcore_barrier()

    def gather_round(pairs):
        cps = [pltpu.async_copy(table.at[idx], dst, sem)
               for table, idx, dst in pairs]
        for cp in cps:
            cp.wait()

    def pass_prefix(j, _):
        offs = [pl.multiple_of(j * (16 * VPI) + u * 16, 16) for u in range(VPI)]
        qs = [qbuf[pl.ds(o, 16)] for o in offs]
        ls = [jnp.zeros((16,), jnp.int32) for _ in range(VPI)]
        for d in range(16):
            for u in range(VPI):
                i = (ls[u] >> 3) + jnp.int32(1 << (15 - d))
                t = plsc.load_gather(pre_v, [i])
                c = qs[u] > t
                ls[u] = ls[u] + jnp.where(c, jnp.int32(1 << (18 - d)),
                                          jnp.int32(0))
        for u in range(VPI):
            lbuf[pl.ds(offs[u], 16)] = ls[u]
            midb[pl.ds(offs[u], 16)] = ls[u] + jnp.int32(4)
        return 0

    def make_mid_pass(step, nxt):
        def body(j, _):
            for u in range(VPI):
                off = pl.multiple_of(j * (16 * VPI) + u * 16, 16)
                q = qbuf[pl.ds(off, 16)]
                l = lbuf[pl.ds(off, 16)]
                t = cmpb[pl.ds(off, 16)]
                l = l + jnp.where(q > t, jnp.int32(step), jnp.int32(0))
                lbuf[pl.ds(off, 16)] = l
                midb[pl.ds(off, 16)] = l + jnp.int32(nxt)
            return 0
        return body

    def pass_final(j, _):
        for u in range(VPI):
            off = pl.multiple_of(j * (16 * VPI) + u * 16, 16)
            q = qbuf[pl.ds(off, 16)]
            l = lbuf[pl.ds(off, 16)]
            t1 = cmpb[pl.ds(off, 16)]
            t0 = cmp2b[pl.ds(off, 16)]
            c = q > t1
            l = l + c.astype(jnp.int32)
            tf = jnp.where(c, t1, t0)
            res = l + (q > tf).astype(jnp.int32)
            midb[pl.ds(off, 16)] = jnp.minimum(res, jnp.int32(N - 1))
        return 0

    def chunk_body(ch, _):
        base = pl.multiple_of(wid * QPW + ch * CHUNK, CHUNK)
        pltpu.sync_copy(q_hbm.at[pl.ds(base, CHUNK)], qbuf)
        lax.fori_loop(0, NITER, pass_prefix, 0)
        gather_round([(sbnd, midb, cmpb)])
        lax.fori_loop(0, NITER, make_mid_pass(4, 2), 0)
        gather_round([(sbnd, midb, cmpb)])
        lax.fori_loop(0, NITER, make_mid_pass(2, 1), 0)
        gather_round([(sbnd, midb, cmpb),
                      (sbnd, lbuf, cmp2b)])
        lax.fori_loop(0, NITER, pass_final, 0)
        gather_round([(val_hbm, midb, obuf)])
        pltpu.sync_copy(obuf, out_hbm.at[pl.ds(base, CHUNK)])
        return 0

    lax.fori_loop(0, NCHUNK, chunk_body, 0)


@jax.jit
def kernel(image, yx_res):
    b, c, h, w = yx_res.shape
    xs = (jnp.arange(w, dtype=jnp.float32) / (w - 1)) * 2.0 - 1.0
    ys = (jnp.arange(h, dtype=jnp.float32) / (h - 1)) * 2.0 - 1.0
    xm = jnp.broadcast_to(xs[None, :], (h, w))
    ym = jnp.broadcast_to(ys[:, None], (h, w))
    bnd = jnp.stack([xm + yx_res[0, 0], ym + yx_res[0, 1]], axis=-1).ravel()
    val = jnp.stack([xm + yx_res[1, 0], ym + yx_res[1, 1]], axis=-1).ravel()
    pre = bnd.reshape(PRE, 8)[:, 0]
    qflat = image.ravel()

    mesh = plsc.VectorSubcoreMesh(core_axis_name="c", subcore_axis_name="s")
    out = pl.kernel(
        _sc_body,
        out_type=jax.ShapeDtypeStruct((NQ,), jnp.float32),
        mesh=mesh,
        compiler_params=pltpu.CompilerParams(needs_layout_passes=False),
        scratch_types=[
            pltpu.VMEM((PRE,), jnp.float32),
            pltpu.VMEM((CHUNK,), jnp.float32),
            pltpu.VMEM((CHUNK,), jnp.int32),
            pltpu.VMEM((CHUNK,), jnp.int32),
            pltpu.VMEM((CHUNK,), jnp.float32),
            pltpu.VMEM((CHUNK,), jnp.float32),
            pltpu.VMEM((CHUNK,), jnp.float32),
            pltpu.VMEM_SHARED((N,), jnp.float32),
            pltpu.SemaphoreType.DMA,
        ],
    )(qflat, bnd, val, pre)
    return out.reshape(image.shape)


# speedup vs baseline: 2.6130x; 2.6083x over previous
import functools

import jax
import jax.numpy as jnp
from jax import lax
from jax.experimental import pallas as pl
from jax.experimental.pallas import tpu as pltpu
from jax.experimental.pallas import tpu_sc as plsc

H = W = 512
N = 2 * H * W
NC = N - N // 8
NQ = 8 * 3 * H * W
NW = 32
QPW = NQ // NW
CHUNK = 4096
NCHUNK = QPW // CHUNK
NVREG = CHUNK // 16
VPI = 4
NITER = NVREG // VPI
PRE = N // 8


def _comp(i):
    return i - (i >> 3) - jnp.int32(1)


def _bucketize_body(q_hbm, bndc_hbm, pre_hbm, res_hbm,
                    pre_v, qbuf, lbuf, midb, idx2b, cmpb, cmp2b, sbnd, sem):
    cid = lax.axis_index("c")
    sid = lax.axis_index("s")
    wid = sid * 2 + cid

    pltpu.sync_copy(pre_hbm, pre_v)

    @pl.when(sid == 0)
    def _stage():
        pltpu.sync_copy(bndc_hbm, sbnd)

    plsc.subcore_barrier()

    def gather_round(pairs):
        cps = [pltpu.async_copy(table.at[idx], dst, sem)
               for table, idx, dst in pairs]
        for cp in cps:
            cp.wait()

    def pass_prefix(j, _):
        offs = [pl.multiple_of(j * (16 * VPI) + u * 16, 16) for u in range(VPI)]
        qs = [qbuf[pl.ds(o, 16)] for o in offs]
        ls = [jnp.zeros((16,), jnp.int32) for _ in range(VPI)]
        for d in range(16):
            for u in range(VPI):
                i = (ls[u] >> 3) + jnp.int32(1 << (15 - d))
                t = plsc.load_gather(pre_v, [i])
                c = qs[u] > t
                ls[u] = ls[u] + jnp.where(c, jnp.int32(1 << (18 - d)),
                                          jnp.int32(0))
        for u in range(VPI):
            lbuf[pl.ds(offs[u], 16)] = ls[u]
            midb[pl.ds(offs[u], 16)] = _comp(ls[u] + jnp.int32(4))
        return 0

    def pass_d16(j, _):
        for u in range(VPI):
            off = pl.multiple_of(j * (16 * VPI) + u * 16, 16)
            q = qbuf[pl.ds(off, 16)]
            l = lbuf[pl.ds(off, 16)]
            t = cmpb[pl.ds(off, 16)]
            l = l + jnp.where(q > t, jnp.int32(4), jnp.int32(0))
            lbuf[pl.ds(off, 16)] = l
            midb[pl.ds(off, 16)] = _comp(l + jnp.int32(2))
        return 0

    def pass_d17(j, _):
        for u in range(VPI):
            off = pl.multiple_of(j * (16 * VPI) + u * 16, 16)
            q = qbuf[pl.ds(off, 16)]
            l = lbuf[pl.ds(off, 16)]
            t = cmpb[pl.ds(off, 16)]
            l = l + jnp.where(q > t, jnp.int32(2), jnp.int32(0))
            lbuf[pl.ds(off, 16)] = l
            midb[pl.ds(off, 16)] = _comp(l + jnp.int32(1))
            m8 = (l & jnp.int32(7)) == jnp.int32(0)
            idx2b[pl.ds(off, 16)] = jnp.where(m8, jnp.int32(0), _comp(l))
        return 0

    def pass_final(j, _):
        for u in range(VPI):
            off = pl.multiple_of(j * (16 * VPI) + u * 16, 16)
            q = qbuf[pl.ds(off, 16)]
            l = lbuf[pl.ds(off, 16)]
            t1 = cmpb[pl.ds(off, 16)]
            t0g = cmp2b[pl.ds(off, 16)]
            m8 = (l & jnp.int32(7)) == jnp.int32(0)
            t_pre = plsc.load_gather(pre_v, [l >> 3])
            t0 = jnp.where(m8, t_pre, t0g)
            c = q > t1
            l = l + c.astype(jnp.int32)
            tf = jnp.where(c, t1, t0)
            res = l + (q > tf).astype(jnp.int32)
            midb[pl.ds(off, 16)] = jnp.minimum(res, jnp.int32(N - 1))
        return 0

    def chunk_body(ch, _):
        base = pl.multiple_of(wid * QPW + ch * CHUNK, CHUNK)
        pltpu.sync_copy(q_hbm.at[pl.ds(base, CHUNK)], qbuf)
        lax.fori_loop(0, NITER, pass_prefix, 0)
        gather_round([(sbnd, midb, cmpb)])
        lax.fori_loop(0, NITER, pass_d16, 0)
        gather_round([(sbnd, midb, cmpb)])
        lax.fori_loop(0, NITER, pass_d17, 0)
        gather_round([(sbnd, midb, cmpb),
                      (sbnd, idx2b, cmp2b)])
        lax.fori_loop(0, NITER, pass_final, 0)
        pltpu.sync_copy(midb, res_hbm.at[pl.ds(base, CHUNK)])
        return 0

    lax.fori_loop(0, NCHUNK, chunk_body, 0)


def _values_body(res_hbm, val_hbm, out_hbm, rbuf, obuf, sval, sem):
    cid = lax.axis_index("c")
    sid = lax.axis_index("s")
    wid = sid * 2 + cid

    @pl.when(sid == 0)
    def _stage():
        pltpu.sync_copy(val_hbm, sval)

    plsc.subcore_barrier()

    def chunk_body(ch, _):
        base = pl.multiple_of(wid * QPW + ch * CHUNK, CHUNK)
        pltpu.sync_copy(res_hbm.at[pl.ds(base, CHUNK)], rbuf)
        pltpu.async_copy(sval.at[rbuf], obuf, sem).wait()
        pltpu.sync_copy(obuf, out_hbm.at[pl.ds(base, CHUNK)])
        return 0

    lax.fori_loop(0, NCHUNK, chunk_body, 0)


@jax.jit
def kernel(image, yx_res):
    b, c, h, w = yx_res.shape
    xs = (jnp.arange(w, dtype=jnp.float32) / (w - 1)) * 2.0 - 1.0
    ys = (jnp.arange(h, dtype=jnp.float32) / (h - 1)) * 2.0 - 1.0
    xm = jnp.broadcast_to(xs[None, :], (h, w))
    ym = jnp.broadcast_to(ys[:, None], (h, w))
    bnd = jnp.stack([xm + yx_res[0, 0], ym + yx_res[0, 1]], axis=-1).ravel()
    val = jnp.stack([xm + yx_res[1, 0], ym + yx_res[1, 1]], axis=-1).ravel()
    bnd8 = bnd.reshape(PRE, 8)
    pre = bnd8[:, 0]
    bndc = bnd8[:, 1:].reshape(NC)
    qflat = image.ravel()

    mesh = plsc.VectorSubcoreMesh(core_axis_name="c", subcore_axis_name="s")
    res = pl.kernel(
        _bucketize_body,
        out_type=jax.ShapeDtypeStruct((NQ,), jnp.int32),
        mesh=mesh,
        compiler_params=pltpu.CompilerParams(needs_layout_passes=False),
        scratch_types=[
            pltpu.VMEM((PRE,), jnp.float32),
            pltpu.VMEM((CHUNK,), jnp.float32),
            pltpu.VMEM((CHUNK,), jnp.int32),
            pltpu.VMEM((CHUNK,), jnp.int32),
            pltpu.VMEM((CHUNK,), jnp.int32),
            pltpu.VMEM((CHUNK,), jnp.float32),
            pltpu.VMEM((CHUNK,), jnp.float32),
            pltpu.VMEM_SHARED((NC,), jnp.float32),
            pltpu.SemaphoreType.DMA,
        ],
    )(qflat, bndc, pre)

    out = pl.kernel(
        _values_body,
        out_type=jax.ShapeDtypeStruct((NQ,), jnp.float32),
        mesh=mesh,
        compiler_params=pltpu.CompilerParams(needs_layout_passes=False),
        scratch_types=[
            pltpu.VMEM((CHUNK,), jnp.int32),
            pltpu.VMEM((CHUNK,), jnp.float32),
            pltpu.VMEM_SHARED((N,), jnp.float32),
            pltpu.SemaphoreType.DMA,
        ],
    )(res, val)
    return out.reshape(image.shape)


# speedup vs baseline: 2.8424x; 1.0878x over previous
import functools

import jax
import jax.numpy as jnp
from jax import lax
from jax.experimental import pallas as pl
from jax.experimental.pallas import tpu as pltpu
from jax.experimental.pallas import tpu_sc as plsc

H = W = 512
N = 2 * H * W
NC = N - N // 8
NQ = 8 * 3 * H * W
NW = 32
QPW = NQ // NW
CHUNK = 4096
NCHUNK = QPW // CHUNK
NVREG = CHUNK // 16
VPI = 4
NITER = NVREG // VPI
PRE = N // 8


def _comp(i):
    return i - (i >> 3) - jnp.int32(1)


def _bucketize_body(q_hbm, bndc_hbm, pre_hbm, res_hbm,
                    pre_v, qbuf, lbuf, midb, idx2b, cmpb, cmp2b, sbnd, sem):
    cid = lax.axis_index("c")
    sid = lax.axis_index("s")
    wid = sid * 2 + cid

    pltpu.sync_copy(pre_hbm, pre_v)

    @pl.when(sid == 0)
    def _stage():
        pltpu.sync_copy(bndc_hbm, sbnd)

    plsc.subcore_barrier()

    def gather_round(pairs):
        cps = [pltpu.async_copy(table.at[idx], dst, sem)
               for table, idx, dst in pairs]
        for cp in cps:
            cp.wait()

    def pass_prefix(j):
        off = pl.multiple_of(j * 16, 16)
        q = qbuf[pl.ds(off, 16)]
        i = jnp.full((16,), 1 << 15, jnp.int32)
        for d in range(15):
            t = plsc.load_gather(pre_v, [i])
            c = q > t
            a = jnp.int32(1 << (14 - d))
            i = i + jnp.where(c, a, -a)
        t = plsc.load_gather(pre_v, [i])
        c = q > t
        p = i + jnp.where(c, jnp.int32(0), jnp.int32(-1))
        lbuf[pl.ds(off, 16)] = p << 3
        midb[pl.ds(off, 16)] = p * jnp.int32(7) + jnp.int32(3)

    def pass_d16(j):
        off = pl.multiple_of(j * 16, 16)
        q = qbuf[pl.ds(off, 16)]
        l = lbuf[pl.ds(off, 16)]
        t = cmpb[pl.ds(off, 16)]
        l = l + jnp.where(q > t, jnp.int32(4), jnp.int32(0))
        lbuf[pl.ds(off, 16)] = l
        midb[pl.ds(off, 16)] = _comp(l + jnp.int32(2))

    def pass_d17(j):
        off = pl.multiple_of(j * 16, 16)
        q = qbuf[pl.ds(off, 16)]
        l = lbuf[pl.ds(off, 16)]
        t = cmpb[pl.ds(off, 16)]
        l = l + jnp.where(q > t, jnp.int32(2), jnp.int32(0))
        lbuf[pl.ds(off, 16)] = l
        midb[pl.ds(off, 16)] = _comp(l + jnp.int32(1))
        m8 = (l & jnp.int32(7)) == jnp.int32(0)
        idx2b[pl.ds(off, 16)] = jnp.where(m8, jnp.int32(0), _comp(l))

    def pass_final(j):
        off = pl.multiple_of(j * 16, 16)
        q = qbuf[pl.ds(off, 16)]
        l = lbuf[pl.ds(off, 16)]
        t1 = cmpb[pl.ds(off, 16)]
        t0g = cmp2b[pl.ds(off, 16)]
        m8 = (l & jnp.int32(7)) == jnp.int32(0)
        t_pre = plsc.load_gather(pre_v, [l >> 3])
        t0 = jnp.where(m8, t_pre, t0g)
        c = q > t1
        l = l + c.astype(jnp.int32)
        tf = jnp.where(c, t1, t0)
        res = l + (q > tf).astype(jnp.int32)
        midb[pl.ds(off, 16)] = jnp.minimum(res, jnp.int32(N - 1))

    def chunk_body(ch, _):
        base = pl.multiple_of(wid * QPW + ch * CHUNK, CHUNK)
        pltpu.sync_copy(q_hbm.at[pl.ds(base, CHUNK)], qbuf)
        plsc.parallel_loop(0, NVREG, unroll=8)(pass_prefix)
        gather_round([(sbnd, midb, cmpb)])
        plsc.parallel_loop(0, NVREG, unroll=8)(pass_d16)
        gather_round([(sbnd, midb, cmpb)])
        plsc.parallel_loop(0, NVREG, unroll=8)(pass_d17)
        gather_round([(sbnd, midb, cmpb),
                      (sbnd, idx2b, cmp2b)])
        plsc.parallel_loop(0, NVREG, unroll=8)(pass_final)
        pltpu.sync_copy(midb, res_hbm.at[pl.ds(base, CHUNK)])
        return 0

    lax.fori_loop(0, NCHUNK, chunk_body, 0)


def _values_body(res_hbm, val_hbm, out_hbm, rbuf, obuf, sval, sem):
    cid = lax.axis_index("c")
    sid = lax.axis_index("s")
    wid = sid * 2 + cid

    @pl.when(sid == 0)
    def _stage():
        pltpu.sync_copy(val_hbm, sval)

    plsc.subcore_barrier()

    def chunk_body(ch, _):
        base = pl.multiple_of(wid * QPW + ch * CHUNK, CHUNK)
        pltpu.sync_copy(res_hbm.at[pl.ds(base, CHUNK)], rbuf)
        pltpu.async_copy(sval.at[rbuf], obuf, sem).wait()
        pltpu.sync_copy(obuf, out_hbm.at[pl.ds(base, CHUNK)])
        return 0

    lax.fori_loop(0, NCHUNK, chunk_body, 0)


@jax.jit
def kernel(image, yx_res):
    b, c, h, w = yx_res.shape
    xs = (jnp.arange(w, dtype=jnp.float32) / (w - 1)) * 2.0 - 1.0
    ys = (jnp.arange(h, dtype=jnp.float32) / (h - 1)) * 2.0 - 1.0
    xm = jnp.broadcast_to(xs[None, :], (h, w))
    ym = jnp.broadcast_to(ys[:, None], (h, w))
    bnd = jnp.stack([xm + yx_res[0, 0], ym + yx_res[0, 1]], axis=-1).ravel()
    val = jnp.stack([xm + yx_res[1, 0], ym + yx_res[1, 1]], axis=-1).ravel()
    bnd8 = bnd.reshape(PRE, 8)
    pre = bnd8[:, 0]
    bndc = bnd8[:, 1:].reshape(NC)
    qflat = image.ravel()

    mesh = plsc.VectorSubcoreMesh(core_axis_name="c", subcore_axis_name="s")
    res = pl.kernel(
        _bucketize_body,
        out_type=jax.ShapeDtypeStruct((NQ,), jnp.int32),
        mesh=mesh,
        compiler_params=pltpu.CompilerParams(needs_layout_passes=False),
        scratch_types=[
            pltpu.VMEM((PRE,), jnp.float32),
            pltpu.VMEM((CHUNK,), jnp.float32),
            pltpu.VMEM((CHUNK,), jnp.int32),
            pltpu.VMEM((CHUNK,), jnp.int32),
            pltpu.VMEM((CHUNK,), jnp.int32),
            pltpu.VMEM((CHUNK,), jnp.float32),
            pltpu.VMEM((CHUNK,), jnp.float32),
            pltpu.VMEM_SHARED((NC,), jnp.float32),
            pltpu.SemaphoreType.DMA,
        ],
    )(qflat, bndc, pre)

    out = pl.kernel(
        _values_body,
        out_type=jax.ShapeDtypeStruct((NQ,), jnp.float32),
        mesh=mesh,
        compiler_params=pltpu.CompilerParams(needs_layout_passes=False),
        scratch_types=[
            pltpu.VMEM((CHUNK,), jnp.int32),
            pltpu.VMEM((CHUNK,), jnp.float32),
            pltpu.VMEM_SHARED((N,), jnp.float32),
            pltpu.SemaphoreType.DMA,
        ],
    )(res, val)
    return out.reshape(image.shape)


# speedup vs baseline: 3.1883x; 1.1217x over previous
import functools

import jax
import jax.numpy as jnp
from jax import lax
from jax.experimental import pallas as pl
from jax.experimental.pallas import tpu as pltpu
from jax.experimental.pallas import tpu_sc as plsc

H = W = 512
N = 2 * H * W
NC = N - N // 8
NQ = 8 * 3 * H * W
NW = 32
QPW = NQ // NW
CHUNK = 4096
NCHUNK = QPW // CHUNK
NVREG = CHUNK // 16
VPI = 4
NITER = NVREG // VPI
PRE = N // 8


def _comp(i):
    return i - (i >> 3) - jnp.int32(1)


def _bucketize_body(q_hbm, bndc_hbm, pre_hbm, res_hbm,
                    pre_v, qbuf, lbuf, midb, idx2b, cmpb, cmp2b, sbnd, sem):
    cid = lax.axis_index("c")
    sid = lax.axis_index("s")
    wid = sid * 2 + cid

    pltpu.sync_copy(pre_hbm, pre_v)

    @pl.when(sid == 0)
    def _stage():
        pltpu.sync_copy(bndc_hbm, sbnd)

    plsc.subcore_barrier()

    def gather_round(pairs):
        cps = [pltpu.async_copy(table.at[idx], dst, sem)
               for table, idx, dst in pairs]
        for cp in cps:
            cp.wait()

    def _hoist(idx):
        return plsc.load_gather(pre_v, [jnp.full((16,), idx, jnp.int32)])

    I0 = 1 << 15
    t_l0 = _hoist(I0)
    t_l1 = [_hoist(I0 + s1 * (1 << 14)) for s1 in (-1, 1)]
    t_l2 = [[_hoist(I0 + s1 * (1 << 14) + s2 * (1 << 13)) for s2 in (-1, 1)]
            for s1 in (-1, 1)]

    def pass_prefix(j):
        off = pl.multiple_of(j * 16, 16)
        q = qbuf[pl.ds(off, 16)]
        c0 = q > t_l0
        i = jnp.full((16,), I0, jnp.int32)
        i = i + jnp.where(c0, jnp.int32(1 << 14), jnp.int32(-(1 << 14)))
        c1 = q > jnp.where(c0, t_l1[1], t_l1[0])
        i = i + jnp.where(c1, jnp.int32(1 << 13), jnp.int32(-(1 << 13)))
        t2 = jnp.where(c0, jnp.where(c1, t_l2[1][1], t_l2[1][0]),
                       jnp.where(c1, t_l2[0][1], t_l2[0][0]))
        c2 = q > t2
        i = i + jnp.where(c2, jnp.int32(1 << 12), jnp.int32(-(1 << 12)))
        for d in range(3, 15):
            t = plsc.load_gather(pre_v, [i])
            c = q > t
            a = jnp.int32(1 << (14 - d))
            i = i + jnp.where(c, a, -a)
        t = plsc.load_gather(pre_v, [i])
        c = q > t
        p = i + jnp.where(c, jnp.int32(0), jnp.int32(-1))
        lbuf[pl.ds(off, 16)] = p << 3
        midb[pl.ds(off, 16)] = p * jnp.int32(7) + jnp.int32(3)

    def pass_d16(j):
        off = pl.multiple_of(j * 16, 16)
        q = qbuf[pl.ds(off, 16)]
        l = lbuf[pl.ds(off, 16)]
        t = cmpb[pl.ds(off, 16)]
        l = l + jnp.where(q > t, jnp.int32(4), jnp.int32(0))
        lbuf[pl.ds(off, 16)] = l
        midb[pl.ds(off, 16)] = _comp(l + jnp.int32(2))

    def pass_d17(j):
        off = pl.multiple_of(j * 16, 16)
        q = qbuf[pl.ds(off, 16)]
        l = lbuf[pl.ds(off, 16)]
        t = cmpb[pl.ds(off, 16)]
        l = l + jnp.where(q > t, jnp.int32(2), jnp.int32(0))
        lbuf[pl.ds(off, 16)] = l
        midb[pl.ds(off, 16)] = _comp(l + jnp.int32(1))
        m8 = (l & jnp.int32(7)) == jnp.int32(0)
        idx2b[pl.ds(off, 16)] = jnp.where(m8, jnp.int32(0), _comp(l))

    def pass_final(j):
        off = pl.multiple_of(j * 16, 16)
        q = qbuf[pl.ds(off, 16)]
        l = lbuf[pl.ds(off, 16)]
        t1 = cmpb[pl.ds(off, 16)]
        t0g = cmp2b[pl.ds(off, 16)]
        m8 = (l & jnp.int32(7)) == jnp.int32(0)
        t_pre = plsc.load_gather(pre_v, [l >> 3])
        t0 = jnp.where(m8, t_pre, t0g)
        c = q > t1
        l = l + c.astype(jnp.int32)
        tf = jnp.where(c, t1, t0)
        res = l + (q > tf).astype(jnp.int32)
        midb[pl.ds(off, 16)] = jnp.minimum(res, jnp.int32(N - 1))

    def chunk_body(ch, _):
        base = pl.multiple_of(wid * QPW + ch * CHUNK, CHUNK)
        pltpu.sync_copy(q_hbm.at[pl.ds(base, CHUNK)], qbuf)
        plsc.parallel_loop(0, NVREG, unroll=16)(pass_prefix)
        gather_round([(sbnd, midb, cmpb)])
        plsc.parallel_loop(0, NVREG, unroll=8)(pass_d16)
        gather_round([(sbnd, midb, cmpb)])
        plsc.parallel_loop(0, NVREG, unroll=8)(pass_d17)
        gather_round([(sbnd, midb, cmpb),
                      (sbnd, idx2b, cmp2b)])
        plsc.parallel_loop(0, NVREG, unroll=8)(pass_final)
        pltpu.sync_copy(midb, res_hbm.at[pl.ds(base, CHUNK)])
        return 0

    lax.fori_loop(0, NCHUNK, chunk_body, 0)


def _values_body(res_hbm, val_hbm, out_hbm, rbuf, obuf, sval, sem):
    cid = lax.axis_index("c")
    sid = lax.axis_index("s")
    wid = sid * 2 + cid

    @pl.when(sid == 0)
    def _stage():
        pltpu.sync_copy(val_hbm, sval)

    plsc.subcore_barrier()

    def chunk_body(ch, _):
        base = pl.multiple_of(wid * QPW + ch * CHUNK, CHUNK)
        pltpu.sync_copy(res_hbm.at[pl.ds(base, CHUNK)], rbuf)
        pltpu.async_copy(sval.at[rbuf], obuf, sem).wait()
        pltpu.sync_copy(obuf, out_hbm.at[pl.ds(base, CHUNK)])
        return 0

    lax.fori_loop(0, NCHUNK, chunk_body, 0)


@jax.jit
def kernel(image, yx_res):
    b, c, h, w = yx_res.shape
    xs = (jnp.arange(w, dtype=jnp.float32) / (w - 1)) * 2.0 - 1.0
    ys = (jnp.arange(h, dtype=jnp.float32) / (h - 1)) * 2.0 - 1.0
    xm = jnp.broadcast_to(xs[None, :], (h, w))
    ym = jnp.broadcast_to(ys[:, None], (h, w))
    bnd = jnp.stack([xm + yx_res[0, 0], ym + yx_res[0, 1]], axis=-1).ravel()
    val = jnp.stack([xm + yx_res[1, 0], ym + yx_res[1, 1]], axis=-1).ravel()
    bnd8 = bnd.reshape(PRE, 8)
    pre = bnd8[:, 0]
    bndc = bnd8[:, 1:].reshape(NC)
    qflat = image.ravel()

    mesh = plsc.VectorSubcoreMesh(core_axis_name="c", subcore_axis_name="s")
    res = pl.kernel(
        _bucketize_body,
        out_type=jax.ShapeDtypeStruct((NQ,), jnp.int32),
        mesh=mesh,
        compiler_params=pltpu.CompilerParams(needs_layout_passes=False),
        scratch_types=[
            pltpu.VMEM((PRE,), jnp.float32),
            pltpu.VMEM((CHUNK,), jnp.float32),
            pltpu.VMEM((CHUNK,), jnp.int32),
            pltpu.VMEM((CHUNK,), jnp.int32),
            pltpu.VMEM((CHUNK,), jnp.int32),
            pltpu.VMEM((CHUNK,), jnp.float32),
            pltpu.VMEM((CHUNK,), jnp.float32),
            pltpu.VMEM_SHARED((NC,), jnp.float32),
            pltpu.SemaphoreType.DMA,
        ],
    )(qflat, bndc, pre)

    out = pl.kernel(
        _values_body,
        out_type=jax.ShapeDtypeStruct((NQ,), jnp.float32),
        mesh=mesh,
        compiler_params=pltpu.CompilerParams(needs_layout_passes=False),
        scratch_types=[
            pltpu.VMEM((CHUNK,), jnp.int32),
            pltpu.VMEM((CHUNK,), jnp.float32),
            pltpu.VMEM_SHARED((N,), jnp.float32),
            pltpu.SemaphoreType.DMA,
        ],
    )(res, val)
    return out.reshape(image.shape)
